# baseline (speedup 1.0000x reference)
import functools

import jax
import jax.numpy as jnp
from jax import lax
from jax.experimental import pallas as pl
from jax.experimental.pallas import tpu as pltpu
from jax.experimental.pallas import tpu_sc as plsc

B_ROWS = 16384
L_LABELS = 200
L_HALF = L_LABELS // 2
C_CLICK = 50
C_HALF = C_CLICK // 2
M_INPUT = B_ROWS * 202
TAB_ROWS = M_INPUT // 128
ROW_BLK = 512
N_BLKS = B_ROWS // ROW_BLK
TOTAL_IDX = B_ROWS * C_CLICK
NUM_WORKERS = 32
ROWS_PER_W = B_ROWS // NUM_WORKERS
WORDS_PER_W = ROWS_PER_W * C_HALF
ELEMS_PER_W = TOTAL_IDX // NUM_WORKERS
SUB_ROWS = 256
N_SUB = ROWS_PER_W // SUB_ROWS
SUB_WORDS = SUB_ROWS * C_HALF
SUB_GROUPS = SUB_WORDS // 16
HI_BASE = SUB_ROWS * 128 - 128


def _popcount32(v):
    m1 = jnp.int32(0x55555555)
    m2 = jnp.int32(0x33333333)
    m4 = jnp.int32(0x0F0F0F0F)
    v = v - (lax.shift_right_logical(v, 1) & m1)
    v = (v & m2) + (lax.shift_right_logical(v, 2) & m2)
    v = (v + lax.shift_right_logical(v, 4)) & m4
    return lax.shift_right_logical(v * jnp.int32(0x01010101), 24)


def _shift_lanes_right(x, sh):
    n = x.shape[1]
    z = jnp.zeros((1, sh), jnp.int32)
    return jnp.concatenate([z, x[:, : n - sh]], axis=1)


def _offsets_body(lp_ref, lcp_ref, w0_ref, addr_ref, carry_ref):
    i = pl.program_id(0)

    @pl.when(i == 0)
    def _():
        carry_ref[0] = 0

    iota32 = lax.broadcasted_iota(jnp.int32, (32, ROW_BLK), 0)
    m31 = jnp.int32(31)
    one = jnp.int32(1)

    def body(j, bm):
        p = lp_ref[pl.ds(j, 1), :]
        w_lo = lax.shift_right_logical(p, 5) & m31
        b_lo = lax.shift_left(one, p & m31)
        w_hi = lax.shift_right_logical(p, 21)
        b_hi = lax.shift_left(one, lax.shift_right_logical(p, 16) & m31)
        u = jnp.where(iota32 == w_lo, b_lo, 0)
        u = u | jnp.where(iota32 == w_hi, b_hi, 0)
        return bm | u

    bm = lax.fori_loop(
        0, L_HALF, body, jnp.zeros((32, ROW_BLK), jnp.int32), unroll=4
    )
    uniq = jnp.sum(_popcount32(bm), axis=0, keepdims=True)
    inc = uniq + 1

    x = inc
    sh = 1
    while sh < ROW_BLK:
        x = x + _shift_lanes_right(x, sh)
        sh *= 2
    excl = x - inc
    carry = carry_ref[0]
    offs = excl + carry
    carry_ref[0] = carry + jnp.sum(inc)
    w0_ref[...] = lax.shift_right_logical(offs, 7)

    lane = lax.broadcasted_iota(jnp.int32, (1, ROW_BLK), 1)
    rm128 = (lane & (SUB_ROWS - 1)) * 128
    o7 = offs & 127
    base = rm128 + o7
    hi16 = jnp.int32(HI_BASE)
    zero = jnp.int32(0)
    lcp = lcp_ref[...]
    t_lo = (lcp & jnp.int32(63)) + o7
    t_hi = lax.shift_right_logical(lcp, 16) + o7
    a_lo = t_lo + rm128 + jnp.where(t_lo >= 128, hi16, zero)
    a_hi = t_hi + rm128 + jnp.where(t_hi >= 128, hi16, zero)
    addr_ref[...] = a_lo | lax.shift_left(a_hi, 16)


def _compute_addr(lab_pack, lc_pack):
    return pl.pallas_call(
        _offsets_body,
        grid=(N_BLKS,),
        in_specs=[
            pl.BlockSpec((L_HALF, ROW_BLK), lambda i: (0, i)),
            pl.BlockSpec((C_HALF, ROW_BLK), lambda i: (0, i)),
        ],
        out_specs=[
            pl.BlockSpec((1, ROW_BLK), lambda i: (0, i)),
            pl.BlockSpec((C_HALF, ROW_BLK), lambda i: (0, i)),
        ],
        out_shape=[
            jax.ShapeDtypeStruct((1, B_ROWS), jnp.int32),
            jax.ShapeDtypeStruct((C_HALF, B_ROWS), jnp.int32),
        ],
        scratch_shapes=[pltpu.SMEM((1,), jnp.int32)],
        compiler_params=pltpu.CompilerParams(
            dimension_semantics=("arbitrary",)
        ),
    )(lab_pack, lc_pack)


def _gather_sc(table, w0, addr_rm):
    mesh = plsc.VectorSubcoreMesh(core_axis_name="c", subcore_axis_name="s")

    @functools.partial(
        pl.kernel,
        out_type=jax.ShapeDtypeStruct((TOTAL_IDX,), jnp.float32),
        mesh=mesh,
        compiler_params=pltpu.CompilerParams(needs_layout_passes=False),
        scratch_types=[
            pltpu.VMEM((ROWS_PER_W,), jnp.int32),
            pltpu.VMEM((ROWS_PER_W,), jnp.int32),
            pltpu.VMEM((WORDS_PER_W,), jnp.int32),
            pltpu.VMEM((2 * SUB_ROWS, 128), jnp.float32),
            pltpu.VMEM((ELEMS_PER_W,), jnp.float32),
            pltpu.SemaphoreType.DMA,
            pltpu.SemaphoreType.DMA,
        ],
    )
    def gather_kernel(
        tab_hbm, w0_hbm, addr_hbm, out_hbm,
        w0_v, w1_v, addr_v, buf_v, val_v, sem0, sem1,
    ):
        wid = lax.axis_index("s") * 2 + lax.axis_index("c")
        rbase = wid * ROWS_PER_W
        pbase = wid * WORDS_PER_W
        ebase = wid * ELEMS_PER_W
        pltpu.sync_copy(w0_hbm.at[pl.ds(rbase, ROWS_PER_W)], w0_v)
        pltpu.sync_copy(addr_hbm.at[pl.ds(pbase, WORDS_PER_W)], addr_v)

        def mk_w1(i, _):
            w1_v[pl.ds(i * 16, 16)] = w0_v[pl.ds(i * 16, 16)] + 1
            return 0

        lax.fori_loop(0, ROWS_PER_W // 16, mk_w1, 0, unroll=8)

        mask16 = jnp.int32(0xFFFF)

        for b in range(N_SUB):
            c0 = pltpu.async_copy(
                tab_hbm.at[w0_v.at[pl.ds(b * SUB_ROWS, SUB_ROWS)]],
                buf_v.at[pl.ds(0, SUB_ROWS)],
                sem0,
            )
            c1 = pltpu.async_copy(
                tab_hbm.at[w1_v.at[pl.ds(b * SUB_ROWS, SUB_ROWS)]],
                buf_v.at[pl.ds(SUB_ROWS, SUB_ROWS)],
                sem1,
            )
            c0.wait()
            c1.wait()

            wbase = b * SUB_WORDS
            vbase = b * SUB_WORDS * 2

            def body(g, _):
                w = addr_v[pl.ds(wbase + g * 16, 16)]
                lo = w & mask16
                hi = lax.shift_right_logical(w, 16)
                for half, off in ((lo, 0), (hi, 16)):
                    row = lax.shift_right_logical(half, 7)
                    col = half & 127
                    val_v[pl.ds(vbase + g * 32 + off, 16)] = plsc.load_gather(
                        buf_v, [row, col]
                    )
                return 0

            lax.fori_loop(0, SUB_GROUPS, body, 0, unroll=8)
        pltpu.sync_copy(val_v, out_hbm.at[pl.ds(ebase, ELEMS_PER_W)])

    return gather_kernel(table, w0, addr_rm)


def _loss_body(g_ref, out_ref):
    x = g_ref[...]
    s = -jnp.log(jax.nn.sigmoid(x) + 1e-8)
    out_ref[0, 0] = jnp.sum(s) * (1.0 / TOTAL_IDX)


def _reduce_loss(gathered2d):
    return pl.pallas_call(
        _loss_body,
        out_shape=jax.ShapeDtypeStruct((1, 1), jnp.float32),
        out_specs=pl.BlockSpec(memory_space=pltpu.SMEM),
    )(gathered2d)


def kernel(input, labels, labels_clicked):
    lab_pack = labels[:, :L_HALF].T | lax.shift_left(
        labels[:, L_HALF:].T, 16
    )
    lc_pack = labels_clicked[:, :C_HALF].T | lax.shift_left(
        labels_clicked[:, C_HALF:].T, 16
    )
    w0, addr = _compute_addr(lab_pack, lc_pack)
    table = input.reshape(TAB_ROWS, 128)
    addr_rm = addr.T.reshape(-1)
    gathered = _gather_sc(table, w0.reshape(-1), addr_rm)
    out = _reduce_loss(gathered.reshape(6400, 128))
    return out[0, 0]


# speedup vs baseline: 1.1075x; 1.1075x over previous
import functools

import jax
import jax.numpy as jnp
from jax import lax
from jax.experimental import pallas as pl
from jax.experimental.pallas import tpu as pltpu
from jax.experimental.pallas import tpu_sc as plsc

B_ROWS = 16384
L_LABELS = 200
L_HALF = L_LABELS // 2
C_CLICK = 50
C_HALF = C_CLICK // 2
M_INPUT = B_ROWS * 202
TAB_ROWS = M_INPUT // 128
ROW_BLK = 512
N_BLKS = B_ROWS // ROW_BLK
TOTAL_IDX = B_ROWS * C_CLICK
NUM_WORKERS = 32
ROWS_PER_W = B_ROWS // NUM_WORKERS
WORDS_PER_W = ROWS_PER_W * C_HALF
ELEMS_PER_W = TOTAL_IDX // NUM_WORKERS
SUB_ROWS = 256
N_SUB = ROWS_PER_W // SUB_ROWS
SUB_WORDS = SUB_ROWS * C_HALF
SUB_GROUPS = SUB_WORDS // 16
HI_BASE = SUB_ROWS * 128 - 128


def _popcount32(v):
    m1 = jnp.int32(0x55555555)
    m2 = jnp.int32(0x33333333)
    m4 = jnp.int32(0x0F0F0F0F)
    v = v - (lax.shift_right_logical(v, 1) & m1)
    v = (v & m2) + (lax.shift_right_logical(v, 2) & m2)
    v = (v + lax.shift_right_logical(v, 4)) & m4
    return lax.shift_right_logical(v * jnp.int32(0x01010101), 24)


def _shift_lanes_right(x, sh):
    n = x.shape[1]
    z = jnp.zeros((1, sh), jnp.int32)
    return jnp.concatenate([z, x[:, : n - sh]], axis=1)


def _offsets_body(labels_ref, lc_ref, w0_ref, addr_ref, carry_ref):
    i = pl.program_id(0)

    @pl.when(i == 0)
    def _():
        carry_ref[0] = 0

    iota32 = lax.broadcasted_iota(jnp.int32, (32, ROW_BLK), 0)

    def body(j, bm):
        lrow = labels_ref[pl.ds(j, 1), :]
        w = lax.shift_right_logical(lrow, 5)
        b = lax.shift_left(jnp.int32(1), lrow & 31)
        return bm | jnp.where(iota32 == w, b, 0)

    bm = lax.fori_loop(
        0, L_LABELS, body, jnp.zeros((32, ROW_BLK), jnp.int32), unroll=8
    )
    uniq = jnp.sum(_popcount32(bm), axis=0, keepdims=True)
    inc = uniq + 1

    x = inc
    sh = 1
    while sh < ROW_BLK:
        x = x + _shift_lanes_right(x, sh)
        sh *= 2
    excl = x - inc
    carry = carry_ref[0]
    offs = excl + carry
    carry_ref[0] = carry + jnp.sum(inc)
    w0_ref[...] = lax.shift_right_logical(offs, 7)

    lane = lax.broadcasted_iota(jnp.int32, (1, ROW_BLK), 1)
    rm128 = (lane & (SUB_ROWS - 1)) * 128
    t = lc_ref[...] + (offs & 127)
    addr = t + rm128 + jnp.where(t >= 128, jnp.int32(HI_BASE), jnp.int32(0))
    addr_ref[...] = addr[:C_HALF, :] | lax.shift_left(addr[C_HALF:, :], 16)


def _compute_addr(labels_t, lc_t):
    return pl.pallas_call(
        _offsets_body,
        grid=(N_BLKS,),
        in_specs=[
            pl.BlockSpec((L_LABELS, ROW_BLK), lambda i: (0, i)),
            pl.BlockSpec((C_CLICK, ROW_BLK), lambda i: (0, i)),
        ],
        out_specs=[
            pl.BlockSpec((1, ROW_BLK), lambda i: (0, i)),
            pl.BlockSpec((C_HALF, ROW_BLK), lambda i: (0, i)),
        ],
        out_shape=[
            jax.ShapeDtypeStruct((1, B_ROWS), jnp.int32),
            jax.ShapeDtypeStruct((C_HALF, B_ROWS), jnp.int32),
        ],
        scratch_shapes=[pltpu.SMEM((1,), jnp.int32)],
        compiler_params=pltpu.CompilerParams(
            dimension_semantics=("arbitrary",)
        ),
    )(labels_t, lc_t)


def _gather_sc(table, w0, addr_rm):
    mesh = plsc.VectorSubcoreMesh(core_axis_name="c", subcore_axis_name="s")

    @functools.partial(
        pl.kernel,
        out_type=jax.ShapeDtypeStruct((TOTAL_IDX,), jnp.float32),
        mesh=mesh,
        compiler_params=pltpu.CompilerParams(needs_layout_passes=False),
        scratch_types=[
            pltpu.VMEM((ROWS_PER_W,), jnp.int32),
            pltpu.VMEM((ROWS_PER_W,), jnp.int32),
            pltpu.VMEM((WORDS_PER_W,), jnp.int32),
            pltpu.VMEM((2 * SUB_ROWS, 128), jnp.float32),
            pltpu.VMEM((ELEMS_PER_W,), jnp.float32),
            pltpu.SemaphoreType.DMA,
            pltpu.SemaphoreType.DMA,
        ],
    )
    def gather_kernel(
        tab_hbm, w0_hbm, addr_hbm, out_hbm,
        w0_v, w1_v, addr_v, buf_v, val_v, sem0, sem1,
    ):
        wid = lax.axis_index("s") * 2 + lax.axis_index("c")
        rbase = wid * ROWS_PER_W
        pbase = wid * WORDS_PER_W
        ebase = wid * ELEMS_PER_W
        pltpu.sync_copy(w0_hbm.at[pl.ds(rbase, ROWS_PER_W)], w0_v)
        pltpu.sync_copy(addr_hbm.at[pl.ds(pbase, WORDS_PER_W)], addr_v)

        def mk_w1(i, _):
            w1_v[pl.ds(i * 16, 16)] = w0_v[pl.ds(i * 16, 16)] + 1
            return 0

        lax.fori_loop(0, ROWS_PER_W // 16, mk_w1, 0, unroll=8)

        mask16 = jnp.int32(0xFFFF)

        for b in range(N_SUB):
            c0 = pltpu.async_copy(
                tab_hbm.at[w0_v.at[pl.ds(b * SUB_ROWS, SUB_ROWS)]],
                buf_v.at[pl.ds(0, SUB_ROWS)],
                sem0,
            )
            c1 = pltpu.async_copy(
                tab_hbm.at[w1_v.at[pl.ds(b * SUB_ROWS, SUB_ROWS)]],
                buf_v.at[pl.ds(SUB_ROWS, SUB_ROWS)],
                sem1,
            )
            c0.wait()
            c1.wait()

            wbase = b * SUB_WORDS
            vbase = b * SUB_WORDS * 2

            def body(g, _):
                w = addr_v[pl.ds(wbase + g * 16, 16)]
                lo = w & mask16
                hi = lax.shift_right_logical(w, 16)
                for half, off in ((lo, 0), (hi, 16)):
                    row = lax.shift_right_logical(half, 7)
                    col = half & 127
                    val_v[pl.ds(vbase + g * 32 + off, 16)] = plsc.load_gather(
                        buf_v, [row, col]
                    )
                return 0

            lax.fori_loop(0, SUB_GROUPS, body, 0, unroll=8)
        pltpu.sync_copy(val_v, out_hbm.at[pl.ds(ebase, ELEMS_PER_W)])

    return gather_kernel(table, w0, addr_rm)


def _loss_body(g_ref, out_ref):
    x = g_ref[...]
    s = -jnp.log(jax.nn.sigmoid(x) + 1e-8)
    out_ref[0, 0] = jnp.sum(s) * (1.0 / TOTAL_IDX)


def _reduce_loss(gathered2d):
    return pl.pallas_call(
        _loss_body,
        out_shape=jax.ShapeDtypeStruct((1, 1), jnp.float32),
        out_specs=pl.BlockSpec(memory_space=pltpu.SMEM),
    )(gathered2d)


def kernel(input, labels, labels_clicked):
    labels_t = labels.T
    lc_t = labels_clicked.T
    w0, addr = _compute_addr(labels_t, lc_t)
    table = input.reshape(TAB_ROWS, 128)
    addr_rm = addr.T.reshape(-1)
    gathered = _gather_sc(table, w0.reshape(-1), addr_rm)
    out = _reduce_loss(gathered.reshape(6400, 128))
    return out[0, 0]


# speedup vs baseline: 1.1362x; 1.0259x over previous
import functools

import jax
import jax.numpy as jnp
from jax import lax
from jax.experimental import pallas as pl
from jax.experimental.pallas import tpu as pltpu
from jax.experimental.pallas import tpu_sc as plsc

B_ROWS = 16384
L_LABELS = 200
L_HALF = L_LABELS // 2
C_CLICK = 50
C_HALF = C_CLICK // 2
M_INPUT = B_ROWS * 202
TAB_ROWS = M_INPUT // 128
ROW_BLK = 512
N_BLKS = B_ROWS // ROW_BLK
TOTAL_IDX = B_ROWS * C_CLICK
NUM_WORKERS = 32
ROWS_PER_W = B_ROWS // NUM_WORKERS
WORDS_PER_W = ROWS_PER_W * C_HALF
ELEMS_PER_W = TOTAL_IDX // NUM_WORKERS
SUB_ROWS = 128
N_SUB = ROWS_PER_W // SUB_ROWS
SUB_WORDS = SUB_ROWS * C_HALF
SUB_GROUPS = SUB_WORDS // 16
HI_BASE = SUB_ROWS * 128 - 128


def _popcount32(v):
    m1 = jnp.int32(0x55555555)
    m2 = jnp.int32(0x33333333)
    m4 = jnp.int32(0x0F0F0F0F)
    v = v - (lax.shift_right_logical(v, 1) & m1)
    v = (v & m2) + (lax.shift_right_logical(v, 2) & m2)
    v = (v + lax.shift_right_logical(v, 4)) & m4
    return lax.shift_right_logical(v * jnp.int32(0x01010101), 24)


def _shift_lanes_right(x, sh):
    n = x.shape[1]
    z = jnp.zeros((1, sh), jnp.int32)
    return jnp.concatenate([z, x[:, : n - sh]], axis=1)


def _offsets_body(labels_ref, lc_ref, w0_ref, addr_ref, carry_ref):
    i = pl.program_id(0)

    @pl.when(i == 0)
    def _():
        carry_ref[0] = 0

    iota32 = lax.broadcasted_iota(jnp.int32, (32, ROW_BLK), 0)

    def body(j, bm):
        lrow = labels_ref[pl.ds(j, 1), :]
        w = lax.shift_right_logical(lrow, 5)
        b = lax.shift_left(jnp.int32(1), lrow & 31)
        return bm | jnp.where(iota32 == w, b, 0)

    bm = lax.fori_loop(
        0, L_LABELS, body, jnp.zeros((32, ROW_BLK), jnp.int32), unroll=8
    )
    uniq = jnp.sum(_popcount32(bm), axis=0, keepdims=True)
    inc = uniq + 1

    x = inc
    sh = 1
    while sh < ROW_BLK:
        x = x + _shift_lanes_right(x, sh)
        sh *= 2
    excl = x - inc
    carry = carry_ref[0]
    offs = excl + carry
    carry_ref[0] = carry + jnp.sum(inc)
    w0_ref[...] = lax.shift_right_logical(offs, 7)

    lane = lax.broadcasted_iota(jnp.int32, (1, ROW_BLK), 1)
    rm128 = (lane & (SUB_ROWS - 1)) * 128
    t = lc_ref[...] + (offs & 127)
    addr = t + rm128 + jnp.where(t >= 128, jnp.int32(HI_BASE), jnp.int32(0))
    addr_ref[...] = addr[:C_HALF, :] | lax.shift_left(addr[C_HALF:, :], 16)


def _compute_addr(labels_t, lc_t):
    return pl.pallas_call(
        _offsets_body,
        grid=(N_BLKS,),
        in_specs=[
            pl.BlockSpec((L_LABELS, ROW_BLK), lambda i: (0, i)),
            pl.BlockSpec((C_CLICK, ROW_BLK), lambda i: (0, i)),
        ],
        out_specs=[
            pl.BlockSpec((1, ROW_BLK), lambda i: (0, i)),
            pl.BlockSpec((C_HALF, ROW_BLK), lambda i: (0, i)),
        ],
        out_shape=[
            jax.ShapeDtypeStruct((1, B_ROWS), jnp.int32),
            jax.ShapeDtypeStruct((C_HALF, B_ROWS), jnp.int32),
        ],
        scratch_shapes=[pltpu.SMEM((1,), jnp.int32)],
        compiler_params=pltpu.CompilerParams(
            dimension_semantics=("arbitrary",)
        ),
    )(labels_t, lc_t)


def _gather_sc(table, w0, addr_rm):
    mesh = plsc.VectorSubcoreMesh(core_axis_name="c", subcore_axis_name="s")

    @functools.partial(
        pl.kernel,
        out_type=jax.ShapeDtypeStruct((TOTAL_IDX,), jnp.float32),
        mesh=mesh,
        compiler_params=pltpu.CompilerParams(needs_layout_passes=False),
        scratch_types=[
            pltpu.VMEM((ROWS_PER_W,), jnp.int32),
            pltpu.VMEM((ROWS_PER_W,), jnp.int32),
            pltpu.VMEM((WORDS_PER_W,), jnp.int32),
            pltpu.VMEM((2 * SUB_ROWS, 128), jnp.float32),
            pltpu.VMEM((2 * SUB_ROWS, 128), jnp.float32),
            pltpu.VMEM((ELEMS_PER_W,), jnp.float32),
            pltpu.SemaphoreType.DMA,
            pltpu.SemaphoreType.DMA,
        ],
    )
    def gather_kernel(
        tab_hbm, w0_hbm, addr_hbm, out_hbm,
        w0_v, w1_v, addr_v, bufa_v, bufb_v, val_v, sem0, sem1,
    ):
        wid = lax.axis_index("s") * 2 + lax.axis_index("c")
        rbase = wid * ROWS_PER_W
        pbase = wid * WORDS_PER_W
        ebase = wid * ELEMS_PER_W
        pltpu.sync_copy(w0_hbm.at[pl.ds(rbase, ROWS_PER_W)], w0_v)
        pltpu.sync_copy(addr_hbm.at[pl.ds(pbase, WORDS_PER_W)], addr_v)

        def mk_w1(i, _):
            w1_v[pl.ds(i * 16, 16)] = w0_v[pl.ds(i * 16, 16)] + 1
            return 0

        lax.fori_loop(0, ROWS_PER_W // 16, mk_w1, 0, unroll=8)

        mask16 = jnp.int32(0xFFFF)
        bufs = (bufa_v, bufb_v)
        sems = (sem0, sem1)

        def fire(b):
            p = b % 2
            buf, sem = bufs[p], sems[p]
            c0 = pltpu.async_copy(
                tab_hbm.at[w0_v.at[pl.ds(b * SUB_ROWS, SUB_ROWS)]],
                buf.at[pl.ds(0, SUB_ROWS)],
                sem,
            )
            c1 = pltpu.async_copy(
                tab_hbm.at[w1_v.at[pl.ds(b * SUB_ROWS, SUB_ROWS)]],
                buf.at[pl.ds(SUB_ROWS, SUB_ROWS)],
                sem,
            )
            return c0, c1

        pending = {0: fire(0)}
        for b in range(N_SUB):
            if b + 1 < N_SUB:
                pending[b + 1] = fire(b + 1)
            h0, h1 = pending.pop(b)
            h0.wait()
            h1.wait()
            buf_v = bufs[b % 2]

            wbase = b * SUB_WORDS
            vbase = b * SUB_WORDS * 2

            def body(g, _, buf_v=buf_v, wbase=wbase, vbase=vbase):
                w = addr_v[pl.ds(wbase + g * 16, 16)]
                lo = w & mask16
                hi = lax.shift_right_logical(w, 16)
                for half, off in ((lo, 0), (hi, 16)):
                    row = lax.shift_right_logical(half, 7)
                    col = half & 127
                    val_v[pl.ds(vbase + g * 32 + off, 16)] = plsc.load_gather(
                        buf_v, [row, col]
                    )
                return 0

            lax.fori_loop(0, SUB_GROUPS, body, 0, unroll=8)
        pltpu.sync_copy(val_v, out_hbm.at[pl.ds(ebase, ELEMS_PER_W)])

    return gather_kernel(table, w0, addr_rm)


def _loss_body(g_ref, out_ref):
    x = g_ref[...]
    s = -jnp.log(jax.nn.sigmoid(x) + 1e-8)
    out_ref[0, 0] = jnp.sum(s) * (1.0 / TOTAL_IDX)


def _reduce_loss(gathered2d):
    return pl.pallas_call(
        _loss_body,
        out_shape=jax.ShapeDtypeStruct((1, 1), jnp.float32),
        out_specs=pl.BlockSpec(memory_space=pltpu.SMEM),
    )(gathered2d)


def kernel(input, labels, labels_clicked):
    labels_t = labels.T
    lc_t = labels_clicked.T
    w0, addr = _compute_addr(labels_t, lc_t)
    table = input.reshape(TAB_ROWS, 128)
    addr_rm = addr.T.reshape(-1)
    gathered = _gather_sc(table, w0.reshape(-1), addr_rm)
    out = _reduce_loss(gathered.reshape(6400, 128))
    return out[0, 0]


# speedup vs baseline: 1.1514x; 1.0134x over previous
import functools

import jax
import jax.numpy as jnp
from jax import lax
from jax.experimental import pallas as pl
from jax.experimental.pallas import tpu as pltpu
from jax.experimental.pallas import tpu_sc as plsc

B_ROWS = 16384
L_LABELS = 200
L_HALF = L_LABELS // 2
C_CLICK = 50
C_HALF = C_CLICK // 2
M_INPUT = B_ROWS * 202
TAB_ROWS = M_INPUT // 128
ROW_BLK = 512
N_BLKS = B_ROWS // ROW_BLK
TOTAL_IDX = B_ROWS * C_CLICK
NUM_WORKERS = 32
ROWS_PER_W = B_ROWS // NUM_WORKERS
WORDS_PER_W = ROWS_PER_W * C_HALF
ELEMS_PER_W = TOTAL_IDX // NUM_WORKERS
SUB_ROWS = 128
N_SUB = ROWS_PER_W // SUB_ROWS
SUB_WORDS = SUB_ROWS * C_HALF
SUB_GROUPS = SUB_WORDS // 16
HI_BASE = SUB_ROWS * 128 - 128


def _popcount32(v):
    m1 = jnp.int32(0x55555555)
    m2 = jnp.int32(0x33333333)
    m4 = jnp.int32(0x0F0F0F0F)
    v = v - (lax.shift_right_logical(v, 1) & m1)
    v = (v & m2) + (lax.shift_right_logical(v, 2) & m2)
    v = (v + lax.shift_right_logical(v, 4)) & m4
    return lax.shift_right_logical(v * jnp.int32(0x01010101), 24)


def _shift_lanes_right(x, sh):
    n = x.shape[1]
    z = jnp.zeros((1, sh), jnp.int32)
    return jnp.concatenate([z, x[:, : n - sh]], axis=1)


def _offsets_body(labels_ref, lc_ref, w0_ref, addr_ref, carry_ref):
    i = pl.program_id(0)

    @pl.when(i == 0)
    def _():
        carry_ref[0] = 0

    iota32 = lax.broadcasted_iota(jnp.int32, (32, ROW_BLK), 0)

    def body(j, bm):
        lrow = labels_ref[pl.ds(j, 1), :]
        w = lax.shift_right_logical(lrow, 5)
        b = lax.shift_left(jnp.int32(1), lrow & 31)
        return bm | jnp.where(iota32 == w, b, 0)

    bm = lax.fori_loop(
        0, L_LABELS, body, jnp.zeros((32, ROW_BLK), jnp.int32), unroll=8
    )
    uniq = jnp.sum(_popcount32(bm), axis=0, keepdims=True)
    inc = uniq + 1

    x = inc
    sh = 1
    while sh < ROW_BLK:
        x = x + _shift_lanes_right(x, sh)
        sh *= 2
    excl = x - inc
    carry = carry_ref[0]
    offs = excl + carry
    carry_ref[0] = carry + jnp.sum(inc)
    w0_ref[...] = lax.shift_right_logical(offs, 7)

    lane = lax.broadcasted_iota(jnp.int32, (1, ROW_BLK), 1)
    rm128 = (lane & (SUB_ROWS - 1)) * 128
    t = lc_ref[...] + (offs & 127)
    addr = t + rm128 + jnp.where(t >= 128, jnp.int32(HI_BASE), jnp.int32(0))
    addr_ref[...] = addr[:C_HALF, :] | lax.shift_left(addr[C_HALF:, :], 16)


def _compute_addr(labels_t, lc_t):
    return pl.pallas_call(
        _offsets_body,
        grid=(N_BLKS,),
        in_specs=[
            pl.BlockSpec((L_LABELS, ROW_BLK), lambda i: (0, i)),
            pl.BlockSpec((C_CLICK, ROW_BLK), lambda i: (0, i)),
        ],
        out_specs=[
            pl.BlockSpec((1, ROW_BLK), lambda i: (0, i)),
            pl.BlockSpec((C_HALF, ROW_BLK), lambda i: (0, i)),
        ],
        out_shape=[
            jax.ShapeDtypeStruct((1, B_ROWS), jnp.int32),
            jax.ShapeDtypeStruct((C_HALF, B_ROWS), jnp.int32),
        ],
        scratch_shapes=[pltpu.SMEM((1,), jnp.int32)],
        compiler_params=pltpu.CompilerParams(
            dimension_semantics=("arbitrary",)
        ),
    )(labels_t, lc_t)


def _gather_sc(table, w0, addr_rm):
    mesh = plsc.VectorSubcoreMesh(core_axis_name="c", subcore_axis_name="s")

    @functools.partial(
        pl.kernel,
        out_type=jax.ShapeDtypeStruct((TOTAL_IDX,), jnp.float32),
        mesh=mesh,
        compiler_params=pltpu.CompilerParams(needs_layout_passes=False),
        scratch_types=[
            pltpu.VMEM((ROWS_PER_W,), jnp.int32),
            pltpu.VMEM((ROWS_PER_W,), jnp.int32),
            pltpu.VMEM((WORDS_PER_W,), jnp.int32),
            pltpu.VMEM((2 * SUB_ROWS, 128), jnp.float32),
            pltpu.VMEM((2 * SUB_ROWS, 128), jnp.float32),
            pltpu.VMEM((ELEMS_PER_W,), jnp.float32),
            pltpu.SemaphoreType.DMA,
            pltpu.SemaphoreType.DMA,
        ],
    )
    def gather_kernel(
        tab_hbm, w0_hbm, addr_hbm, out_hbm,
        w0_v, w1_v, addr_v, bufa_v, bufb_v, val_v, sem0, sem1,
    ):
        wid = lax.axis_index("s") * 2 + lax.axis_index("c")
        rbase = wid * ROWS_PER_W
        pbase = wid * WORDS_PER_W
        ebase = wid * ELEMS_PER_W
        pltpu.sync_copy(w0_hbm.at[pl.ds(rbase, ROWS_PER_W)], w0_v)

        def mk_w1(i, _):
            w1_v[pl.ds(i * 16, 16)] = w0_v[pl.ds(i * 16, 16)] + 1
            return 0

        lax.fori_loop(0, ROWS_PER_W // 16, mk_w1, 0, unroll=8)

        mask16 = jnp.int32(0xFFFF)
        bufs = (bufa_v, bufb_v)
        sems = (sem0, sem1)

        def fire(b):
            p = b % 2
            buf, sem = bufs[p], sems[p]
            c0 = pltpu.async_copy(
                tab_hbm.at[w0_v.at[pl.ds(b * SUB_ROWS, SUB_ROWS)]],
                buf.at[pl.ds(0, SUB_ROWS)],
                sem,
            )
            c1 = pltpu.async_copy(
                tab_hbm.at[w1_v.at[pl.ds(b * SUB_ROWS, SUB_ROWS)]],
                buf.at[pl.ds(SUB_ROWS, SUB_ROWS)],
                sem,
            )
            return c0, c1

        pending = {0: fire(0)}
        pltpu.sync_copy(addr_hbm.at[pl.ds(pbase, WORDS_PER_W)], addr_v)
        for b in range(N_SUB):
            if b + 1 < N_SUB:
                pending[b + 1] = fire(b + 1)
            h0, h1 = pending.pop(b)
            h0.wait()
            h1.wait()
            buf_v = bufs[b % 2]

            wbase = b * SUB_WORDS
            vbase = b * SUB_WORDS * 2

            def body(g, _, buf_v=buf_v, wbase=wbase, vbase=vbase):
                w = addr_v[pl.ds(wbase + g * 16, 16)]
                lo = w & mask16
                hi = lax.shift_right_logical(w, 16)
                for half, off in ((lo, 0), (hi, 16)):
                    row = lax.shift_right_logical(half, 7)
                    col = half & 127
                    val_v[pl.ds(vbase + g * 32 + off, 16)] = plsc.load_gather(
                        buf_v, [row, col]
                    )
                return 0

            lax.fori_loop(0, SUB_GROUPS, body, 0, unroll=8)
        pltpu.sync_copy(val_v, out_hbm.at[pl.ds(ebase, ELEMS_PER_W)])

    return gather_kernel(table, w0, addr_rm)


def _loss_body(g_ref, out_ref):
    x = g_ref[...]
    s = -jnp.log(jax.nn.sigmoid(x) + 1e-8)
    out_ref[0, 0] = jnp.sum(s) * (1.0 / TOTAL_IDX)


def _reduce_loss(gathered2d):
    return pl.pallas_call(
        _loss_body,
        out_shape=jax.ShapeDtypeStruct((1, 1), jnp.float32),
        out_specs=pl.BlockSpec(memory_space=pltpu.SMEM),
    )(gathered2d)


def kernel(input, labels, labels_clicked):
    labels_t = labels.T
    lc_t = labels_clicked.T
    w0, addr = _compute_addr(labels_t, lc_t)
    table = input.reshape(TAB_ROWS, 128)
    addr_rm = addr.T.reshape(-1)
    gathered = _gather_sc(table, w0.reshape(-1), addr_rm)
    out = _reduce_loss(gathered.reshape(6400, 128))
    return out[0, 0]
